# baseline (speedup 1.0000x reference)
import functools

import jax
import jax.numpy as jnp
from jax import lax
from jax.experimental import pallas as pl
from jax.experimental.pallas import tpu as pltpu
from jax.experimental.pallas import tpu_sc as plsc

N_VERTS = 100000
SIZES = (56, 28, 14, 7)
CHANS = (64, 128, 256, 512)
COL_OFF = (3, 67, 195, 451)
OUT_D = 963
TAB_W = 968

NUM_CORES = 2
NUM_SUBCORES = 16
NUM_WORKERS = NUM_CORES * NUM_SUBCORES
LANES = 16

CHUNK = 32
NSPLIT = 4
NUM_CHUNKS = N_VERTS // CHUNK
ITERS_PER_WORKER = (NUM_CHUNKS + NUM_WORKERS - 1) // NUM_WORKERS

S0 = 56
ZERO_CELL = S0 * S0
TAB_ROWS = 3168
TAB_CHUNKS = TAB_ROWS // CHUNK

_SC_PARAMS = pltpu.CompilerParams(
    use_tc_tiling_on_sc=False, needs_layout_passes=False
)


def _make_table(feat, img_size):
    chans = feat.shape[0]
    s2 = img_size * img_size

    def body(x_ref, o_ref):
        o_ref[...] = jnp.concatenate(
            [x_ref[...].T, jnp.zeros((8, chans), jnp.float32)], axis=0
        )

    return pl.pallas_call(
        body,
        out_shape=jax.ShapeDtypeStruct((s2 + 8, chans), jnp.float32),
    )(feat.reshape(chans, s2))


def _build_body(t0, t1, t2, t3, tab, idx0, idx1, idx2, idx3, g0, g1, g2, g3,
                tbuf, sem):
    wid = lax.axis_index("s") * NUM_CORES + lax.axis_index("c")
    tables = (t0, t1, t2, t3)
    idxbufs = (idx0, idx1, idx2, idx3)
    gbufs = (g0, g1, g2, g3)
    lane = lax.iota(jnp.int32, LANES)

    def chunk_body(ci, _):
        cid = ci * NUM_WORKERS + wid

        @pl.when(cid < TAB_CHUNKS)
        def _():
            base = cid * CHUNK
            for g in range(CHUNK // LANES):
                r = lane + (base + g * LANES)
                i = r // S0
                j = r - i * S0
                p0 = jnp.minimum(r, 3136)
                p1 = jnp.minimum((i >> 1) * 28 + (j >> 1), 784)
                p2 = jnp.minimum((i >> 2) * 14 + (j >> 2), 196)
                p3 = jnp.minimum((i >> 3) * 7 + (j >> 3), 49)
                for lvl, p in enumerate((p0, p1, p2, p3)):
                    idxbufs[lvl][pl.ds(g * LANES, LANES)] = p

            copies = [
                pltpu.async_copy(tables[lvl].at[idxbufs[lvl]], gbufs[lvl], sem)
                for lvl in range(4)
            ]
            for cp in copies:
                cp.wait()

            def row_body(r, _):
                rv = jnp.broadcast_to(r, (LANES,))
                for lvl in range(4):
                    gb = gbufs[lvl]
                    for k in range(CHANS[lvl] // LANES):
                        v = gb[r, pl.ds(k * LANES, LANES)]
                        cols = lane + (COL_OFF[lvl] + k * LANES)
                        plsc.store_scatter(tbuf, [rv, cols], v)

            lax.fori_loop(0, CHUNK, row_body, None)
            pltpu.sync_copy(tbuf, tab.at[pl.ds(base, CHUNK)])

    lax.fori_loop(0, (TAB_CHUNKS + NUM_WORKERS - 1) // NUM_WORKERS,
                  chunk_body, None)


def _vertex_math(inbuf, g):
    lane = lax.iota(jnp.int32, LANES)
    rows = lane + g * LANES
    c0 = jnp.zeros((LANES,), jnp.int32)
    in0 = plsc.load_gather(inbuf, [rows, c0])
    in1 = plsc.load_gather(inbuf, [rows, c0 + 1])
    in2 = plsc.load_gather(inbuf, [rows, c0 + 2])

    h = 248.0 * (in1 / in2) + 111.5
    w = 248.0 * (in0 / (-in2)) + 111.5
    h = jnp.minimum(jnp.maximum(h, 0.0), 223.0)
    w = jnp.minimum(jnp.maximum(w, 0.0), 223.0)

    oks = []
    for s in SIZES:
        x = h * (s / 224.0)
        y = w * (s / 224.0)
        xi = x.astype(jnp.int32)
        yi = y.astype(jnp.int32)
        x2 = jnp.minimum(
            xi + (x > xi.astype(jnp.float32)).astype(jnp.int32), s - 1
        )
        y2 = jnp.minimum(
            yi + (y > yi.astype(jnp.float32)).astype(jnp.int32), s - 1
        )
        oks.append((x2 > xi) & (y2 > yi))
        if s == S0:
            cell = xi * S0 + yi

    all_zero = ~(oks[0] | oks[1] | oks[2] | oks[3])
    all_ok = oks[0] & oks[1] & oks[2] & oks[3]
    cell = jnp.minimum(jnp.maximum(cell, 0), ZERO_CELL - 1)
    idx = jnp.where(all_zero, ZERO_CELL, cell)
    nonconform = ~(all_zero | all_ok)
    return rows, in0, in1, in2, oks, all_zero, nonconform, idx


def _gather_body(tab, inp, out,
                 idxb0, idxb1, inb0, inb1, inb2, inb3, gb0, gb1, ob0, ob1,
                 gs0, gs1, os0, os1, is0, is1, is2, is3):
    wid = lax.axis_index("s") * NUM_CORES + lax.axis_index("c")
    lane = lax.iota(jnp.int32, LANES)
    zeros_f = jnp.zeros((LANES,), jnp.float32)
    idxbs = (idxb0, idxb1)
    inbufs = (inb0, inb1, inb2, inb3)
    gbufs = (gb0, gb1)
    obufs = (ob0, ob1)
    gsems = (gs0, gs1)
    osems = (os0, os1)
    isems = (is0, is1, is2, is3)

    def in_slice(cid):
        return inp.at[pl.ds(cid * CHUNK, CHUNK)]

    pltpu.async_copy(in_slice(wid), inbufs[0], isems[0])

    def outer(o, _):
        for b in range(4):
            ib = b % 2
            pb = (b + 1) % 2
            i = o * 4 + b
            cid = i * NUM_WORKERS + wid
            cid_prev = cid - NUM_WORKERS

            @pl.when(cid < NUM_CHUNKS)
            def _(b=b, ib=ib, cid=cid):
                pltpu.make_async_copy(in_slice(cid), inbufs[b], isems[b]).wait()
                for g in range(CHUNK // LANES):
                    idx = _vertex_math(inbufs[b], g)[-1]
                    idxbs[ib][pl.ds(g * LANES, LANES)] = idx
                for q in range(NSPLIT):
                    qs = q * (CHUNK // NSPLIT)
                    pltpu.async_copy(
                        tab.at[idxbs[ib].at[pl.ds(qs, CHUNK // NSPLIT)]],
                        gbufs[ib].at[pl.ds(qs, CHUNK // NSPLIT)],
                        gsems[ib],
                    )

                @pl.when(cid + NUM_WORKERS < NUM_CHUNKS)
                def _():
                    pltpu.async_copy(
                        in_slice(cid + NUM_WORKERS),
                        inbufs[(b + 1) % 4],
                        isems[(b + 1) % 4],
                    )

            @pl.when((cid_prev >= 0) & (cid_prev < NUM_CHUNKS))
            def _(b=b, pb=pb, cid_prev=cid_prev):
                gbuf = gbufs[pb]
                obuf = obufs[pb]

                @pl.when(cid_prev >= 2 * NUM_WORKERS)
                def _():
                    pltpu.make_async_copy(
                        obuf, out.at[pl.ds(0, CHUNK)], osems[pb]
                    ).wait()

                pltpu.make_async_copy(
                    tab.at[pl.ds(0, CHUNK)], gbuf, gsems[pb]
                ).wait()

                def row_copy(r, _):
                    for k in range(960 // LANES):
                        obuf[r, pl.ds(k * LANES, LANES)] = gbuf[
                            r, pl.ds(k * LANES, LANES)
                        ]

                lax.fori_loop(0, CHUNK, row_copy, None)

                inprev = inbufs[(b + 3) % 4]
                for g in range(CHUNK // LANES):
                    rows, in0, in1, in2, oks, all_zero, nonconform, _ = (
                        _vertex_math(inprev, g)
                    )
                    c0 = jnp.zeros((LANES,), jnp.int32)
                    for c in (960, 961, 962):
                        cc = c0 + c
                        v = plsc.load_gather(gbuf, [rows, cc])
                        plsc.store_scatter(obuf, [rows, cc], v)
                    plsc.store_scatter(obuf, [rows, c0], in0)
                    plsc.store_scatter(obuf, [rows, c0 + 1], in1)
                    plsc.store_scatter(obuf, [rows, c0 + 2], in2)

                    n_bad = jnp.sum(nonconform.astype(jnp.int32))

                    @pl.when(n_bad > 0)
                    def _(rows=rows, oks=oks, all_zero=all_zero):
                        for lvl in range(4):
                            bad = (~oks[lvl]) & (~all_zero)

                            def col_body(k, _, rows=rows, bad=bad, lvl=lvl):
                                cols = jnp.broadcast_to(
                                    COL_OFF[lvl] + k, (LANES,)
                                )
                                plsc.store_scatter(
                                    obuf, [rows, cols], zeros_f, mask=bad
                                )

                            lax.fori_loop(0, CHANS[lvl], col_body, None)

                pltpu.async_copy(
                    obuf, out.at[pl.ds(cid_prev * CHUNK, CHUNK)], osems[pb]
                )

    lax.fori_loop(0, 25, outer, None)

    for j in range(2):
        pltpu.make_async_copy(
            obufs[j], out.at[pl.ds(0, CHUNK)], osems[j]
        ).wait()


@jax.jit
def kernel(img_feat0, img_feat1, img_feat2, img_feat3, input):
    tables = [
        _make_table(f, s)
        for f, s in zip((img_feat0, img_feat1, img_feat2, img_feat3), SIZES)
    ]

    mesh = plsc.VectorSubcoreMesh(core_axis_name="c", subcore_axis_name="s")
    fused = pl.kernel(
        _build_body,
        out_type=jax.ShapeDtypeStruct((TAB_ROWS, TAB_W), jnp.float32),
        mesh=mesh,
        compiler_params=_SC_PARAMS,
        scratch_types=[
            pltpu.VMEM((CHUNK,), jnp.int32),
            pltpu.VMEM((CHUNK,), jnp.int32),
            pltpu.VMEM((CHUNK,), jnp.int32),
            pltpu.VMEM((CHUNK,), jnp.int32),
            pltpu.VMEM((CHUNK, CHANS[0]), jnp.float32),
            pltpu.VMEM((CHUNK, CHANS[1]), jnp.float32),
            pltpu.VMEM((CHUNK, CHANS[2]), jnp.float32),
            pltpu.VMEM((CHUNK, CHANS[3]), jnp.float32),
            pltpu.VMEM((CHUNK, TAB_W), jnp.float32),
            pltpu.SemaphoreType.DMA,
        ],
    )(*tables)

    sc_call = pl.kernel(
        _gather_body,
        out_type=jax.ShapeDtypeStruct((N_VERTS, OUT_D), jnp.float32),
        mesh=mesh,
        compiler_params=_SC_PARAMS,
        scratch_types=[
            pltpu.VMEM((CHUNK,), jnp.int32),
            pltpu.VMEM((CHUNK,), jnp.int32),
            pltpu.VMEM((CHUNK, 3), jnp.float32),
            pltpu.VMEM((CHUNK, 3), jnp.float32),
            pltpu.VMEM((CHUNK, 3), jnp.float32),
            pltpu.VMEM((CHUNK, 3), jnp.float32),
            pltpu.VMEM((CHUNK, TAB_W), jnp.float32),
            pltpu.VMEM((CHUNK, TAB_W), jnp.float32),
            pltpu.VMEM((CHUNK, OUT_D), jnp.float32),
            pltpu.VMEM((CHUNK, OUT_D), jnp.float32),
            pltpu.SemaphoreType.DMA,
            pltpu.SemaphoreType.DMA,
            pltpu.SemaphoreType.DMA,
            pltpu.SemaphoreType.DMA,
            pltpu.SemaphoreType.DMA,
            pltpu.SemaphoreType.DMA,
            pltpu.SemaphoreType.DMA,
            pltpu.SemaphoreType.DMA,
        ],
    )
    return sc_call(fused, input)


# speedup vs baseline: 2.5537x; 2.5537x over previous
import functools

import jax
import jax.numpy as jnp
from jax import lax
from jax.experimental import pallas as pl
from jax.experimental.pallas import tpu as pltpu
from jax.experimental.pallas import tpu_sc as plsc

N_VERTS = 100000
SIZES = (56, 28, 14, 7)
CHANS = (64, 128, 256, 512)
COL_OFF = (3, 67, 195, 451)
OUT_D = 963
TAB_W = 968

NUM_CORES = 2
NUM_SUBCORES = 16
NUM_WORKERS = NUM_CORES * NUM_SUBCORES
LANES = 16

CHUNK = 32
NUM_CHUNKS = N_VERTS // CHUNK
ITERS_PER_WORKER = (NUM_CHUNKS + NUM_WORKERS - 1) // NUM_WORKERS

S0 = 56
ZERO_CELL = S0 * S0
TAB_ROWS = 3168
TAB_CHUNKS = TAB_ROWS // CHUNK

_SC_PARAMS = pltpu.CompilerParams(
    use_tc_tiling_on_sc=False, needs_layout_passes=False
)


def _make_table(feat, img_size):
    chans = feat.shape[0]
    s2 = img_size * img_size

    def body(x_ref, o_ref):
        o_ref[...] = jnp.concatenate(
            [x_ref[...].T, jnp.zeros((8, chans), jnp.float32)], axis=0
        )

    return pl.pallas_call(
        body,
        out_shape=jax.ShapeDtypeStruct((s2 + 8, chans), jnp.float32),
    )(feat.reshape(chans, s2))


def _build_body(t0, t1, t2, t3, tab, idx0, idx1, idx2, idx3, g0, g1, g2, g3,
                tbuf, sem):
    wid = lax.axis_index("s") * NUM_CORES + lax.axis_index("c")
    tables = (t0, t1, t2, t3)
    idxbufs = (idx0, idx1, idx2, idx3)
    gbufs = (g0, g1, g2, g3)
    lane = lax.iota(jnp.int32, LANES)

    def chunk_body(ci, _):
        cid = ci * NUM_WORKERS + wid

        @pl.when(cid < TAB_CHUNKS)
        def _():
            base = cid * CHUNK
            for g in range(CHUNK // LANES):
                r = lane + (base + g * LANES)
                i = r // S0
                j = r - i * S0
                p0 = jnp.minimum(r, 3136)
                p1 = jnp.minimum((i >> 1) * 28 + (j >> 1), 784)
                p2 = jnp.minimum((i >> 2) * 14 + (j >> 2), 196)
                p3 = jnp.minimum((i >> 3) * 7 + (j >> 3), 49)
                for lvl, p in enumerate((p0, p1, p2, p3)):
                    idxbufs[lvl][pl.ds(g * LANES, LANES)] = p

            copies = [
                pltpu.async_copy(tables[lvl].at[idxbufs[lvl]], gbufs[lvl], sem)
                for lvl in range(4)
            ]
            for cp in copies:
                cp.wait()

            def row_body(r, _):
                rv = jnp.broadcast_to(r, (LANES,))
                for lvl in range(4):
                    gb = gbufs[lvl]
                    for k in range(CHANS[lvl] // LANES):
                        v = gb[r, pl.ds(k * LANES, LANES)]
                        cols = lane + (COL_OFF[lvl] + k * LANES)
                        plsc.store_scatter(tbuf, [rv, cols], v)

            lax.fori_loop(0, CHUNK, row_body, None)
            pltpu.sync_copy(tbuf, tab.at[pl.ds(base, CHUNK)])

    lax.fori_loop(0, (TAB_CHUNKS + NUM_WORKERS - 1) // NUM_WORKERS,
                  chunk_body, None)


def _vertex_math(inbuf, g):
    lane = lax.iota(jnp.int32, LANES)
    rows = lane + g * LANES
    c0 = jnp.zeros((LANES,), jnp.int32)
    in0 = plsc.load_gather(inbuf, [rows, c0])
    in1 = plsc.load_gather(inbuf, [rows, c0 + 1])
    in2 = plsc.load_gather(inbuf, [rows, c0 + 2])

    h = 248.0 * (in1 / in2) + 111.5
    w = 248.0 * (in0 / (-in2)) + 111.5
    h = jnp.minimum(jnp.maximum(h, 0.0), 223.0)
    w = jnp.minimum(jnp.maximum(w, 0.0), 223.0)

    oks = []
    for s in SIZES:
        x = h * (s / 224.0)
        y = w * (s / 224.0)
        xi = x.astype(jnp.int32)
        yi = y.astype(jnp.int32)
        x2 = jnp.minimum(
            xi + (x > xi.astype(jnp.float32)).astype(jnp.int32), s - 1
        )
        y2 = jnp.minimum(
            yi + (y > yi.astype(jnp.float32)).astype(jnp.int32), s - 1
        )
        oks.append((x2 > xi) & (y2 > yi))
        if s == S0:
            cell = xi * S0 + yi

    all_zero = ~(oks[0] | oks[1] | oks[2] | oks[3])
    all_ok = oks[0] & oks[1] & oks[2] & oks[3]
    cell = jnp.minimum(jnp.maximum(cell, 0), ZERO_CELL - 1)
    idx = jnp.where(all_zero, ZERO_CELL, cell)
    nonconform = ~(all_zero | all_ok)
    return rows, in0, in1, in2, oks, all_zero, nonconform, idx


def _gather_body(tab, inp, out,
                 ixa0, ixa1, ixa2, ixa3, ixb0, ixb1, ixb2, ixb3,
                 inb0, inb1, inb2, inb3, gb0, gb1, ob0, ob1,
                 gs0, gs1, os0, os1, is0, is1, is2, is3):
    wid = lax.axis_index("s") * NUM_CORES + lax.axis_index("c")
    lane = lax.iota(jnp.int32, LANES)
    zeros_f = jnp.zeros((LANES,), jnp.float32)
    idxbs = ((ixa0, ixa1, ixa2, ixa3), (ixb0, ixb1, ixb2, ixb3))
    inbufs = (inb0, inb1, inb2, inb3)
    gbufs = (gb0, gb1)
    obufs = (ob0, ob1)
    gsems = (gs0, gs1)
    osems = (os0, os1)
    isems = (is0, is1, is2, is3)

    def in_slice(cid):
        return inp.at[pl.ds(cid * CHUNK, CHUNK)]

    zrow = jnp.zeros((LANES,), jnp.float32)
    for j in range(2):
        for k in range(960 // LANES):
            gbufs[j][CHUNK, pl.ds(k * LANES, LANES)] = zrow
        gbufs[j][CHUNK, pl.ds(TAB_W - LANES, LANES)] = zrow

    pltpu.async_copy(in_slice(wid), inbufs[0], isems[0])

    def outer(o, _):
        for b in range(4):
            ib = b % 2
            pb = (b + 1) % 2
            i = o * 4 + b
            cid = i * NUM_WORKERS + wid
            cid_prev = cid - NUM_WORKERS

            @pl.when(cid < NUM_CHUNKS)
            def _(b=b, ib=ib, cid=cid):
                pltpu.make_async_copy(in_slice(cid), inbufs[b], isems[b]).wait()
                k_tot = jnp.int32(0)
                for g in range(CHUNK // LANES):
                    m = _vertex_math(inbufs[b], g)
                    all_zero, idx = m[5], m[7]
                    nz = ~all_zero
                    nzi = nz.astype(jnp.int32)
                    pos = plsc.cumsum(nzi) - 1 + k_tot
                    for q, sz in enumerate((8, 16, 24, 32)):
                        plsc.store_scatter(
                            idxbs[ib][q], [pos], idx, mask=nz & (pos < sz)
                        )
                    k_tot = k_tot + jnp.sum(nzi)
                for q, (lo, sz) in enumerate(((0, 8), (8, 16), (16, 24), (24, 32))):
                    @pl.when((k_tot > lo) & (k_tot <= sz))
                    def _(q=q, sz=sz, ib=ib):
                        pltpu.async_copy(
                            tab.at[idxbs[ib][q]],
                            gbufs[ib].at[pl.ds(0, sz)],
                            gsems[ib],
                        )

                @pl.when(cid + NUM_WORKERS < NUM_CHUNKS)
                def _():
                    pltpu.async_copy(
                        in_slice(cid + NUM_WORKERS),
                        inbufs[(b + 1) % 4],
                        isems[(b + 1) % 4],
                    )

            @pl.when((cid_prev >= 0) & (cid_prev < NUM_CHUNKS))
            def _(b=b, pb=pb, cid_prev=cid_prev):
                gbuf = gbufs[pb]
                obuf = obufs[pb]

                @pl.when(cid_prev >= 2 * NUM_WORKERS)
                def _():
                    pltpu.make_async_copy(
                        obuf, out.at[pl.ds(0, CHUNK)], osems[pb]
                    ).wait()

                inprev = inbufs[(b + 3) % 4]
                maths = [_vertex_math(inprev, g) for g in range(CHUNK // LANES)]
                k_tot = jnp.int32(0)
                srcs = []
                for m in maths:
                    nz = ~m[5]
                    nzi = nz.astype(jnp.int32)
                    pos = plsc.cumsum(nzi) - 1 + k_tot
                    srcs.append(jnp.where(nz, pos, CHUNK))
                    k_tot = k_tot + jnp.sum(nzi)

                for lo, sz in ((0, 8), (8, 16), (16, 24), (24, 32)):
                    @pl.when((k_tot > lo) & (k_tot <= sz))
                    def _(sz=sz, pb=pb):
                        pltpu.make_async_copy(
                            tab.at[pl.ds(0, sz)],
                            gbufs[pb].at[pl.ds(0, sz)],
                            gsems[pb],
                        ).wait()

                for g in range(CHUNK // LANES):
                    rows, in0, in1, in2, oks, all_zero, nonconform, _ = maths[g]
                    src = srcs[g]
                    c0 = jnp.zeros((LANES,), jnp.int32)
                    for k in range(960 // LANES):
                        cols = lane + (3 + k * LANES)
                        v = plsc.load_gather(gbuf, [src, cols])
                        plsc.store_scatter(obuf, [rows, cols], v)
                    plsc.store_scatter(obuf, [rows, c0], in0)
                    plsc.store_scatter(obuf, [rows, c0 + 1], in1)
                    plsc.store_scatter(obuf, [rows, c0 + 2], in2)

                    n_bad = jnp.sum(nonconform.astype(jnp.int32))

                    @pl.when(n_bad > 0)
                    def _(rows=rows, oks=oks, all_zero=all_zero):
                        for lvl in range(4):
                            bad = (~oks[lvl]) & (~all_zero)

                            def col_body(k, _, rows=rows, bad=bad, lvl=lvl):
                                cols = jnp.broadcast_to(
                                    COL_OFF[lvl] + k, (LANES,)
                                )
                                plsc.store_scatter(
                                    obuf, [rows, cols], zeros_f, mask=bad
                                )

                            lax.fori_loop(0, CHANS[lvl], col_body, None)

                pltpu.async_copy(
                    obuf, out.at[pl.ds(cid_prev * CHUNK, CHUNK)], osems[pb]
                )

    lax.fori_loop(0, 25, outer, None)

    for j in range(2):
        pltpu.make_async_copy(
            obufs[j], out.at[pl.ds(0, CHUNK)], osems[j]
        ).wait()


@jax.jit
def kernel(img_feat0, img_feat1, img_feat2, img_feat3, input):
    tables = [
        _make_table(f, s)
        for f, s in zip((img_feat0, img_feat1, img_feat2, img_feat3), SIZES)
    ]

    mesh = plsc.VectorSubcoreMesh(core_axis_name="c", subcore_axis_name="s")
    fused = pl.kernel(
        _build_body,
        out_type=jax.ShapeDtypeStruct((TAB_ROWS, TAB_W), jnp.float32),
        mesh=mesh,
        compiler_params=_SC_PARAMS,
        scratch_types=[
            pltpu.VMEM((CHUNK,), jnp.int32),
            pltpu.VMEM((CHUNK,), jnp.int32),
            pltpu.VMEM((CHUNK,), jnp.int32),
            pltpu.VMEM((CHUNK,), jnp.int32),
            pltpu.VMEM((CHUNK, CHANS[0]), jnp.float32),
            pltpu.VMEM((CHUNK, CHANS[1]), jnp.float32),
            pltpu.VMEM((CHUNK, CHANS[2]), jnp.float32),
            pltpu.VMEM((CHUNK, CHANS[3]), jnp.float32),
            pltpu.VMEM((CHUNK, TAB_W), jnp.float32),
            pltpu.SemaphoreType.DMA,
        ],
    )(*tables)

    sc_call = pl.kernel(
        _gather_body,
        out_type=jax.ShapeDtypeStruct((N_VERTS, OUT_D), jnp.float32),
        mesh=mesh,
        compiler_params=_SC_PARAMS,
        scratch_types=[
            pltpu.VMEM((8,), jnp.int32),
            pltpu.VMEM((16,), jnp.int32),
            pltpu.VMEM((24,), jnp.int32),
            pltpu.VMEM((32,), jnp.int32),
            pltpu.VMEM((8,), jnp.int32),
            pltpu.VMEM((16,), jnp.int32),
            pltpu.VMEM((24,), jnp.int32),
            pltpu.VMEM((32,), jnp.int32),
            pltpu.VMEM((CHUNK, 3), jnp.float32),
            pltpu.VMEM((CHUNK, 3), jnp.float32),
            pltpu.VMEM((CHUNK, 3), jnp.float32),
            pltpu.VMEM((CHUNK, 3), jnp.float32),
            pltpu.VMEM((CHUNK + 1, TAB_W), jnp.float32),
            pltpu.VMEM((CHUNK + 1, TAB_W), jnp.float32),
            pltpu.VMEM((CHUNK, OUT_D), jnp.float32),
            pltpu.VMEM((CHUNK, OUT_D), jnp.float32),
            pltpu.SemaphoreType.DMA,
            pltpu.SemaphoreType.DMA,
            pltpu.SemaphoreType.DMA,
            pltpu.SemaphoreType.DMA,
            pltpu.SemaphoreType.DMA,
            pltpu.SemaphoreType.DMA,
            pltpu.SemaphoreType.DMA,
            pltpu.SemaphoreType.DMA,
        ],
    )
    return sc_call(fused, input)
